# baseline (speedup 1.0000x reference)
import functools

import jax
import jax.numpy as jnp
from jax import lax
from jax.experimental import pallas as pl
from jax.experimental.pallas import tpu as pltpu

H = 8
NEG_INF = float("-inf")


def _fused_kernel(q_ref, qpos_ref, x_ref, xpos_ref, mask_ref,
                  ln_g_ref, ln_b_ref, pW_ref, pb_ref, fW_ref, fb_ref,
                  alpha_ref,
                  out_ref, sim_ref,
                  keyt_ref, qn_ref, rmax_ref, rg_ref,
                  *, bn, nb, q_len, c):
    n = pl.program_id(1)
    d = c // H

    @pl.when(n == 0)
    def _prologue():
        qv = q_ref[0]
        m = jnp.mean(qv, axis=-1, keepdims=True)
        v = jnp.mean((qv - m) ** 2, axis=-1, keepdims=True)
        qn = (qv - m) / jnp.sqrt(v + 1e-5) * ln_g_ref[0] + ln_b_ref[0]
        qn_ref[...] = qn
        q2 = qn + qpos_ref[0]
        nrm = jnp.sqrt(jnp.sum(q2 * q2, axis=-1, keepdims=True))
        keyt_ref[...] = q2 / jnp.maximum(nrm, 1e-12)
        rmax_ref[...] = jnp.full((H, q_len), NEG_INF, jnp.float32)

    x2 = x_ref[0] + xpos_ref[0]
    nrm = jnp.sqrt(jnp.sum(x2 * x2, axis=-1, keepdims=True))
    xs = x2 / jnp.maximum(nrm, 1e-12)

    keyt = keyt_ref[...]
    iota_n = lax.broadcasted_iota(jnp.int32, (bn, q_len), 0)
    for h in range(H):
        ks_h = keyt[:, h * d:(h + 1) * d]
        xs_h = xs[:, h * d:(h + 1) * d]
        s = lax.dot_general(
            xs_h, ks_h, (((1,), (1,)), ((), ())),
            preferred_element_type=jnp.float32)
        mh = mask_ref[h].T
        sm = jnp.where(mh, NEG_INF, s)
        sim_ref[0, h] = sm
        bmax = jnp.max(sm, axis=0)
        bidx = jnp.argmax(sm, axis=0).astype(jnp.int32)
        onehot = jnp.where(bidx[None, :] == iota_n, 1.0, 0.0)
        cand = lax.dot_general(
            xs_h, onehot, (((0,), (0,)), ((), ())),
            preferred_element_type=jnp.float32)
        take = jnp.logical_or(bmax > rmax_ref[h, :], n == 0)
        rmax_ref[h, :] = jnp.where(take, bmax, rmax_ref[h, :])
        rg_ref[h] = jnp.where(take[None, :], cand, rg_ref[h])

    @pl.when(n == nb - 1)
    def _epilogue():
        query_g = rg_ref[...].reshape(c, q_len).T
        kt = keyt_ref[...]
        proj = lax.dot_general(
            query_g * kt, pW_ref[...], (((1,), (1,)), ((), ())),
            preferred_element_type=jnp.float32) + pb_ref[0]
        nrm2 = jnp.sqrt(jnp.sum(proj * proj, axis=0, keepdims=True))
        out1 = proj / jnp.maximum(nrm2, 1e-12) * alpha_ref[0] + query_g
        out2 = lax.dot_general(
            out1, fW_ref[...], (((1,), (1,)), ((), ())),
            preferred_element_type=jnp.float32) + fb_ref[0]
        out_ref[0] = out2 + qn_ref[...]


@jax.jit
def _run(q, x, query_pos, x_pos, attn_mask, ln_g, ln_b, proj_W, proj_b,
         final_W, final_b, alpha):
    B, Q, C = q.shape
    N = x.shape[1]
    BN = 512
    NB = N // BN
    D = C // H

    kern = functools.partial(_fused_kernel, bn=BN, nb=NB, q_len=Q, c=C)
    out, sim = pl.pallas_call(
        kern,
        grid=(B, NB),
        in_specs=[
            pl.BlockSpec((1, Q, C), lambda b, n: (b, 0, 0)),
            pl.BlockSpec((1, Q, C), lambda b, n: (b, 0, 0)),
            pl.BlockSpec((1, BN, C), lambda b, n: (b, n, 0)),
            pl.BlockSpec((1, BN, C), lambda b, n: (b, n, 0)),
            pl.BlockSpec((H, Q, BN), lambda b, n: (b, 0, n)),
            pl.BlockSpec((1, C), lambda b, n: (0, 0)),
            pl.BlockSpec((1, C), lambda b, n: (0, 0)),
            pl.BlockSpec((C, C), lambda b, n: (0, 0)),
            pl.BlockSpec((1, C), lambda b, n: (0, 0)),
            pl.BlockSpec((C, C), lambda b, n: (0, 0)),
            pl.BlockSpec((1, C), lambda b, n: (0, 0)),
            pl.BlockSpec((1, C), lambda b, n: (0, 0)),
        ],
        out_specs=[
            pl.BlockSpec((1, Q, C), lambda b, n: (b, 0, 0)),
            pl.BlockSpec((1, H, BN, Q), lambda b, n: (b, 0, n, 0)),
        ],
        out_shape=[
            jax.ShapeDtypeStruct((B, Q, C), jnp.float32),
            jax.ShapeDtypeStruct((B, H, N, Q), jnp.float32),
        ],
        scratch_shapes=[
            pltpu.VMEM((Q, C), jnp.float32),
            pltpu.VMEM((Q, C), jnp.float32),
            pltpu.VMEM((H, Q), jnp.float32),
            pltpu.VMEM((H, D, Q), jnp.float32),
        ],
        compiler_params=pltpu.CompilerParams(
            dimension_semantics=("arbitrary", "arbitrary")),
    )(q, query_pos, x, x_pos, attn_mask,
      ln_g, ln_b, proj_W, proj_b, final_W, final_b, alpha)
    return out, sim


def kernel(q, x, query_pos, x_pos, attn_mask, need_weights, ln_g, ln_b,
           proj_W, proj_b, final_W, final_b, alpha):
    return _run(q, x, query_pos, x_pos, attn_mask,
                ln_g.reshape(1, -1), ln_b.reshape(1, -1),
                proj_W, proj_b.reshape(1, -1),
                final_W, final_b.reshape(1, -1),
                alpha.reshape(1, -1))


# speedup vs baseline: 1.0397x; 1.0397x over previous
import functools

import jax
import jax.numpy as jnp
from jax import lax
from jax.experimental import pallas as pl
from jax.experimental.pallas import tpu as pltpu

H = 8
NEG_INF = float("-inf")


def _fused_kernel(q_ref, qpos_ref, x_ref, xpos_ref, mask_ref,
                  ln_g_ref, ln_b_ref, pW_ref, pb_ref, fW_ref, fb_ref,
                  alpha_ref,
                  out_ref, sim_ref,
                  xs_ref, keyt_ref, qn_ref, rmax_ref, ridx_ref,
                  *, bn, nb, n_total, q_len, c):
    n = pl.program_id(1)
    d = c // H

    @pl.when(n == 0)
    def _prologue():
        qv = q_ref[0]
        m = jnp.mean(qv, axis=-1, keepdims=True)
        v = jnp.mean((qv - m) ** 2, axis=-1, keepdims=True)
        qn = (qv - m) / jnp.sqrt(v + 1e-5) * ln_g_ref[0] + ln_b_ref[0]
        qn_ref[...] = qn
        q2 = qn + qpos_ref[0]
        nrm = jnp.sqrt(jnp.sum(q2 * q2, axis=-1, keepdims=True))
        keyt_ref[...] = q2 / jnp.maximum(nrm, 1e-12)
        rmax_ref[...] = jnp.full((H, q_len), NEG_INF, jnp.float32)
        ridx_ref[...] = jnp.zeros((H, q_len), jnp.int32)

    x2 = x_ref[0] + xpos_ref[0]
    nrm = jnp.sqrt(jnp.sum(x2 * x2, axis=-1, keepdims=True))
    xs = x2 / jnp.maximum(nrm, 1e-12)
    xs_ref[pl.ds(n * bn, bn), :] = xs

    keyt = keyt_ref[...]
    for h in range(H):
        ks_h = keyt[:, h * d:(h + 1) * d]
        xs_h = xs[:, h * d:(h + 1) * d]
        s = lax.dot_general(
            xs_h, ks_h, (((1,), (1,)), ((), ())),
            preferred_element_type=jnp.float32)
        mh = mask_ref[h].T
        sm = jnp.where(mh, NEG_INF, s)
        sim_ref[0, h] = sm
        bmax = jnp.max(sm, axis=0)
        bidx = jnp.argmax(sm, axis=0).astype(jnp.int32) + n * bn
        better = bmax > rmax_ref[h, :]
        rmax_ref[h, :] = jnp.where(better, bmax, rmax_ref[h, :])
        ridx_ref[h, :] = jnp.where(better, bidx, ridx_ref[h, :])

    @pl.when(n == nb - 1)
    def _epilogue():
        xs_all = xs_ref[...]
        iota_n = lax.broadcasted_iota(jnp.int32, (q_len, n_total), 1)
        parts = []
        for h in range(H):
            onehot = jnp.where(ridx_ref[h, :][:, None] == iota_n, 1.0, 0.0)
            g_h = lax.dot_general(
                onehot, xs_all[:, h * d:(h + 1) * d],
                (((1,), (0,)), ((), ())),
                preferred_element_type=jnp.float32)
            parts.append(g_h)
        query_g = jnp.concatenate(parts, axis=1)

        kt = keyt_ref[...]
        proj = lax.dot_general(
            query_g * kt, pW_ref[...], (((1,), (1,)), ((), ())),
            preferred_element_type=jnp.float32) + pb_ref[0]
        nrm2 = jnp.sqrt(jnp.sum(proj * proj, axis=0, keepdims=True))
        out1 = proj / jnp.maximum(nrm2, 1e-12) * alpha_ref[0] + query_g
        out2 = lax.dot_general(
            out1, fW_ref[...], (((1,), (1,)), ((), ())),
            preferred_element_type=jnp.float32) + fb_ref[0]
        out_ref[0] = out2 + qn_ref[...]


@jax.jit
def _run(q, x, query_pos, x_pos, attn_mask, ln_g, ln_b, proj_W, proj_b,
         final_W, final_b, alpha):
    B, Q, C = q.shape
    N = x.shape[1]
    BN = 1024
    NB = N // BN

    kern = functools.partial(_fused_kernel, bn=BN, nb=NB, n_total=N,
                             q_len=Q, c=C)
    out, sim = pl.pallas_call(
        kern,
        grid=(B, NB),
        in_specs=[
            pl.BlockSpec((1, Q, C), lambda b, n: (b, 0, 0)),
            pl.BlockSpec((1, Q, C), lambda b, n: (b, 0, 0)),
            pl.BlockSpec((1, BN, C), lambda b, n: (b, n, 0)),
            pl.BlockSpec((1, BN, C), lambda b, n: (b, n, 0)),
            pl.BlockSpec((H, Q, BN), lambda b, n: (b, 0, n)),
            pl.BlockSpec((1, C), lambda b, n: (0, 0)),
            pl.BlockSpec((1, C), lambda b, n: (0, 0)),
            pl.BlockSpec((C, C), lambda b, n: (0, 0)),
            pl.BlockSpec((1, C), lambda b, n: (0, 0)),
            pl.BlockSpec((C, C), lambda b, n: (0, 0)),
            pl.BlockSpec((1, C), lambda b, n: (0, 0)),
            pl.BlockSpec((1, C), lambda b, n: (0, 0)),
        ],
        out_specs=[
            pl.BlockSpec((1, Q, C), lambda b, n: (b, 0, 0)),
            pl.BlockSpec((1, H, BN, Q), lambda b, n: (b, 0, n, 0)),
        ],
        out_shape=[
            jax.ShapeDtypeStruct((B, Q, C), jnp.float32),
            jax.ShapeDtypeStruct((B, H, N, Q), jnp.float32),
        ],
        scratch_shapes=[
            pltpu.VMEM((N, C), jnp.float32),
            pltpu.VMEM((Q, C), jnp.float32),
            pltpu.VMEM((Q, C), jnp.float32),
            pltpu.VMEM((H, Q), jnp.float32),
            pltpu.VMEM((H, Q), jnp.int32),
        ],
        compiler_params=pltpu.CompilerParams(
            dimension_semantics=("arbitrary", "arbitrary")),
    )(q, query_pos, x, x_pos, attn_mask,
      ln_g, ln_b, proj_W, proj_b, final_W, final_b, alpha)
    return out, sim


def kernel(q, x, query_pos, x_pos, attn_mask, need_weights, ln_g, ln_b,
           proj_W, proj_b, final_W, final_b, alpha):
    return _run(q, x, query_pos, x_pos, attn_mask,
                ln_g.reshape(1, -1), ln_b.reshape(1, -1),
                proj_W, proj_b.reshape(1, -1),
                final_W, final_b.reshape(1, -1),
                alpha.reshape(1, -1))


# speedup vs baseline: 1.5417x; 1.4828x over previous
import functools

import jax
import jax.numpy as jnp
from jax import lax
from jax.experimental import pallas as pl
from jax.experimental.pallas import tpu as pltpu

H = 8
NEG_INF = float("-inf")


def _fused_kernel(q_ref, qpos_ref, x_ref, xpos_ref, mask_ref,
                  ln_g_ref, ln_b_ref, pW_ref, pb_ref, fW_ref, fb_ref,
                  alpha_ref,
                  out_ref, sim_ref,
                  xs_ref, keyt_ref, qn_ref, rmax_ref, ridx_ref,
                  *, bn, nb, n_total, q_len, c):
    n = pl.program_id(1)
    d = c // H

    @pl.when(n == 0)
    def _prologue():
        qv = q_ref[0]
        m = jnp.mean(qv, axis=-1, keepdims=True)
        v = jnp.mean((qv - m) ** 2, axis=-1, keepdims=True)
        qn = (qv - m) / jnp.sqrt(v + 1e-5) * ln_g_ref[0] + ln_b_ref[0]
        qn_ref[...] = qn
        q2 = qn + qpos_ref[0]
        nrm = jnp.sqrt(jnp.sum(q2 * q2, axis=-1, keepdims=True))
        keyt_ref[...] = q2 / jnp.maximum(nrm, 1e-12)
        rmax_ref[...] = jnp.full((H, q_len), NEG_INF, jnp.float32)
        ridx_ref[...] = jnp.zeros((H, q_len), jnp.int32)

    x2 = x_ref[0] + xpos_ref[0]
    nrm = jnp.sqrt(jnp.sum(x2 * x2, axis=-1, keepdims=True))
    xs = x2 / jnp.maximum(nrm, 1e-12)
    xs_ref[pl.ds(n * bn, bn), :] = xs

    keyt = keyt_ref[...]
    for h in range(H):
        ks_h = keyt[:, h * d:(h + 1) * d]
        xs_h = xs[:, h * d:(h + 1) * d]
        s = lax.dot_general(
            xs_h, ks_h, (((1,), (1,)), ((), ())),
            preferred_element_type=jnp.float32)
        mh = mask_ref[h].T
        sm = jnp.where(mh != 0, NEG_INF, s)
        sim_ref[0, h] = sm
        bmax = jnp.max(sm, axis=0)
        bidx = jnp.argmax(sm, axis=0).astype(jnp.int32) + n * bn
        better = bmax > rmax_ref[h, :]
        rmax_ref[h, :] = jnp.where(better, bmax, rmax_ref[h, :])
        ridx_ref[h, :] = jnp.where(better, bidx, ridx_ref[h, :])

    @pl.when(n == nb - 1)
    def _epilogue():
        xs_all = xs_ref[...]
        iota_n = lax.broadcasted_iota(jnp.int32, (q_len, n_total), 1)
        parts = []
        for h in range(H):
            onehot = jnp.where(ridx_ref[h, :][:, None] == iota_n, 1.0, 0.0)
            g_h = lax.dot_general(
                onehot, xs_all[:, h * d:(h + 1) * d],
                (((1,), (0,)), ((), ())),
                preferred_element_type=jnp.float32)
            parts.append(g_h)
        query_g = jnp.concatenate(parts, axis=1)

        kt = keyt_ref[...]
        proj = lax.dot_general(
            query_g * kt, pW_ref[...], (((1,), (1,)), ((), ())),
            preferred_element_type=jnp.float32) + pb_ref[0]
        nrm2 = jnp.sqrt(jnp.sum(proj * proj, axis=0, keepdims=True))
        out1 = proj / jnp.maximum(nrm2, 1e-12) * alpha_ref[0] + query_g
        out2 = lax.dot_general(
            out1, fW_ref[...], (((1,), (1,)), ((), ())),
            preferred_element_type=jnp.float32) + fb_ref[0]
        out_ref[0] = out2 + qn_ref[...]


@jax.jit
def _run(q, x, query_pos, x_pos, attn_mask, ln_g, ln_b, proj_W, proj_b,
         final_W, final_b, alpha):
    B, Q, C = q.shape
    N = x.shape[1]
    BN = 1024
    NB = N // BN

    mask_i8 = attn_mask.view(jnp.int8)

    kern = functools.partial(_fused_kernel, bn=BN, nb=NB, n_total=N,
                             q_len=Q, c=C)
    out, sim = pl.pallas_call(
        kern,
        grid=(B, NB),
        in_specs=[
            pl.BlockSpec((1, Q, C), lambda b, n: (b, 0, 0)),
            pl.BlockSpec((1, Q, C), lambda b, n: (b, 0, 0)),
            pl.BlockSpec((1, BN, C), lambda b, n: (b, n, 0)),
            pl.BlockSpec((1, BN, C), lambda b, n: (b, n, 0)),
            pl.BlockSpec((H, Q, BN), lambda b, n: (b, 0, n)),
            pl.BlockSpec((1, C), lambda b, n: (0, 0)),
            pl.BlockSpec((1, C), lambda b, n: (0, 0)),
            pl.BlockSpec((C, C), lambda b, n: (0, 0)),
            pl.BlockSpec((1, C), lambda b, n: (0, 0)),
            pl.BlockSpec((C, C), lambda b, n: (0, 0)),
            pl.BlockSpec((1, C), lambda b, n: (0, 0)),
            pl.BlockSpec((1, C), lambda b, n: (0, 0)),
        ],
        out_specs=[
            pl.BlockSpec((1, Q, C), lambda b, n: (b, 0, 0)),
            pl.BlockSpec((1, H, BN, Q), lambda b, n: (b, 0, n, 0)),
        ],
        out_shape=[
            jax.ShapeDtypeStruct((B, Q, C), jnp.float32),
            jax.ShapeDtypeStruct((B, H, N, Q), jnp.float32),
        ],
        scratch_shapes=[
            pltpu.VMEM((N, C), jnp.float32),
            pltpu.VMEM((Q, C), jnp.float32),
            pltpu.VMEM((Q, C), jnp.float32),
            pltpu.VMEM((H, Q), jnp.float32),
            pltpu.VMEM((H, Q), jnp.int32),
        ],
        compiler_params=pltpu.CompilerParams(
            dimension_semantics=("arbitrary", "arbitrary")),
    )(q, query_pos, x, x_pos, mask_i8,
      ln_g, ln_b, proj_W, proj_b, final_W, final_b, alpha)
    return out, sim


def kernel(q, x, query_pos, x_pos, attn_mask, need_weights, ln_g, ln_b,
           proj_W, proj_b, final_W, final_b, alpha):
    return _run(q, x, query_pos, x_pos, attn_mask,
                ln_g.reshape(1, -1), ln_b.reshape(1, -1),
                proj_W, proj_b.reshape(1, -1),
                final_W, final_b.reshape(1, -1),
                alpha.reshape(1, -1))
